# baseline (speedup 1.0000x reference)
import dataclasses
import functools

import jax
import jax.numpy as jnp
from jax import lax
from jax.experimental import pallas as pl
from jax.experimental.pallas import tpu as pltpu
from jax.experimental.pallas import tpu_sc as plsc

B = 16384
D = 64
V = 1000000
NC = 2
NS = 16
L = 16
NW = NC * NS
BPW = B // NW
CH = 64
NCHUNK = BPW // CH
NGC = CH // L
DU = 8
VB_ROWS = 7816


def _glove_body(ci_hbm, ti_hbm, ec_hbm, et_hbm, bc_hbm, bt_hbm,
                co_hbm, we_hbm, ev_hbm, eu_hbm, vb_hbm,
                ub_hbm, out_hbm,
                ci_v, ti_v, ec_v, et_v, bc_v, bt_v, co_v, we_v,
                ce0, ce1, te0, te1, cb0, cb1, tb0, tb1,
                red_shared, red_v, tot_v, sem0, sem1):
    cid = lax.axis_index("c")
    sid = lax.axis_index("s")
    wid = cid * NS + sid

    ce = (ce0, ce1)
    te = (te0, te1)
    cb = (cb0, cb1)
    tb = (tb0, tb1)
    sems = (sem0, sem1)

    pltpu.sync_copy(ci_hbm.at[wid], ci_v)
    pltpu.sync_copy(ti_hbm.at[wid], ti_v)
    pltpu.sync_copy(ec_hbm.at[wid], ec_v)
    pltpu.sync_copy(et_hbm.at[wid], et_v)
    pltpu.sync_copy(bc_hbm.at[wid], bc_v)
    pltpu.sync_copy(bt_hbm.at[wid], bt_v)
    pltpu.sync_copy(co_hbm.at[wid], co_v)
    pltpu.sync_copy(we_hbm.at[wid], we_v)

    def fire(c, b):
        return [
            pltpu.async_copy(ev_hbm.at[ec_v.at[c]], ce[b], sems[b]),
            pltpu.async_copy(eu_hbm.at[et_v.at[c]], te[b], sems[b]),
            pltpu.async_copy(vb_hbm.at[bc_v.at[c]], cb[b], sems[b]),
            pltpu.async_copy(ub_hbm.at[bt_v.at[c]], tb[b], sems[b]),
        ]

    lane = lax.iota(jnp.int32, L)

    def compute(c, b, lacc):
        def group_body(g, acc):
            s16 = pl.ds(g * L, L)
            rc = ci_v[c, s16]
            rt = ti_v[c, s16]
            kpos = g * L + lane
            col_c = (rc & 1) << 6
            col_t = (rt & 1) << 6

            def d_body(dd, ip):
                d0 = dd * DU
                for k in range(DU):
                    cv = plsc.load_gather(ce[b], [kpos, col_c + (d0 + k)])
                    tv = plsc.load_gather(te[b], [kpos, col_t + (d0 + k)])
                    ip = ip + cv * tv
                return ip

            ip = lax.fori_loop(0, D // DU, d_body,
                               jnp.zeros((L,), jnp.float32))
            cbv = plsc.load_gather(cb[b], [kpos, rc & 127])
            tbv = plsc.load_gather(tb[b], [kpos, rt & 127])
            cov = co_v[pl.ds(c * CH + g * L, L)]
            wev = we_v[pl.ds(c * CH + g * L, L)]
            e = ip + cbv + tbv - cov
            return acc + wev * e * e

        return lax.fori_loop(0, NGC, group_body, lacc)

    cps = fire(0, 0)
    lacc = jnp.zeros((L,), jnp.float32)
    for c in range(NCHUNK):
        b = c % 2
        if c + 1 < NCHUNK:
            ncps = fire(c + 1, (c + 1) % 2)
        else:
            ncps = None
        for cp in cps:
            cp.wait()
        lacc = compute(c, b, lacc)
        cps = ncps

    for i in range(128 // L):
        tot_v[pl.ds(i * L, L)] = lacc if i == 0 else jnp.zeros((L,), jnp.float32)
    pltpu.sync_copy(tot_v, red_shared.at[sid])
    plsc.subcore_barrier()

    @pl.when(sid == 0)
    def _():
        pltpu.sync_copy(red_shared, red_v)
        s = jnp.zeros((L,), jnp.float32)
        for i in range(NS):
            s = s + red_v[i, pl.ds(0, L)]
        total = jnp.sum(s)
        for i in range(128 // L):
            tot_v[pl.ds(i * L, L)] = jnp.full((L,), total, dtype=jnp.float32)
        pltpu.sync_copy(tot_v, out_hbm.at[cid])


_cp = pltpu.CompilerParams()
if "needs_layout_passes" in pltpu.CompilerParams.__dataclass_fields__:
    _cp = dataclasses.replace(_cp, needs_layout_passes=False)

_glove_call = functools.partial(
    pl.kernel,
    compiler_params=_cp,
    out_type=jax.ShapeDtypeStruct((NC, 128), jnp.float32),
    mesh=plsc.VectorSubcoreMesh(core_axis_name="c", subcore_axis_name="s"),
    scratch_types=[
        pltpu.VMEM((NCHUNK, CH), jnp.int32),
        pltpu.VMEM((NCHUNK, CH), jnp.int32),
        pltpu.VMEM((NCHUNK, CH), jnp.int32),
        pltpu.VMEM((NCHUNK, CH), jnp.int32),
        pltpu.VMEM((NCHUNK, CH), jnp.int32),
        pltpu.VMEM((NCHUNK, CH), jnp.int32),
        pltpu.VMEM((BPW,), jnp.float32),
        pltpu.VMEM((BPW,), jnp.float32),
        pltpu.VMEM((CH, 128), jnp.float32),
        pltpu.VMEM((CH, 128), jnp.float32),
        pltpu.VMEM((CH, 128), jnp.float32),
        pltpu.VMEM((CH, 128), jnp.float32),
        pltpu.VMEM((CH, 128), jnp.float32),
        pltpu.VMEM((CH, 128), jnp.float32),
        pltpu.VMEM((CH, 128), jnp.float32),
        pltpu.VMEM((CH, 128), jnp.float32),
        pltpu.VMEM_SHARED((NS, 128), jnp.float32),
        pltpu.VMEM((NS, 128), jnp.float32),
        pltpu.VMEM((128,), jnp.float32),
        pltpu.SemaphoreType.DMA,
        pltpu.SemaphoreType.DMA,
    ],
)(_glove_body)


def kernel(center_words, target_words, coocs, weighting, emb_v, emb_u,
           v_bias, u_bias):
    ci = center_words.reshape(NW, NCHUNK, CH).astype(jnp.int32)
    ti = target_words.reshape(NW, NCHUNK, CH).astype(jnp.int32)
    ec = ci >> 1
    et = ti >> 1
    bc = ci >> 7
    bt = ti >> 7
    co = coocs.reshape(NW, BPW)
    we = weighting.reshape(NW, BPW)
    ev = emb_v.reshape(V // 2, 128)
    eu = emb_u.reshape(V // 2, 128)
    vb = jnp.pad(v_bias.reshape(-1), (0, VB_ROWS * 128 - V)).reshape(VB_ROWS, 128)
    ub = jnp.pad(u_bias.reshape(-1), (0, VB_ROWS * 128 - V)).reshape(VB_ROWS, 128)
    out = _glove_call(ci, ti, ec, et, bc, bt, co, we, ev, eu, vb, ub)
    return out[0, 0] + out[1, 0]


# speedup vs baseline: 1.3768x; 1.3768x over previous
import dataclasses
import functools

import jax
import jax.numpy as jnp
from jax import lax
from jax.experimental import pallas as pl
from jax.experimental.pallas import tpu as pltpu
from jax.experimental.pallas import tpu_sc as plsc

B = 16384
D = 64
V = 1000000
NC = 2
NS = 16
L = 16
NW = NC * NS
BPW = B // NW
CH = 16
NCHUNK = BPW // CH
VB_ROWS = 7816


def _glove_body(ci_hbm, ti_hbm, bc_hbm, bt_hbm,
                co_hbm, we_hbm, ev_hbm, eu_hbm, vb_hbm,
                ub_hbm, out_hbm,
                ci_v, ti_v, bc_v, bt_v, co_v, we_v,
                ce0, ce1, te0, te1, cb0, cb1, tb0, tb1, p_v,
                red_shared, red_v, tot_v, sem0, sem1):
    cid = lax.axis_index("c")
    sid = lax.axis_index("s")
    wid = cid * NS + sid

    ce = (ce0, ce1)
    te = (te0, te1)
    cb = (cb0, cb1)
    tb = (tb0, tb1)
    sems = (sem0, sem1)

    pltpu.sync_copy(ci_hbm.at[wid], ci_v)
    pltpu.sync_copy(ti_hbm.at[wid], ti_v)
    pltpu.sync_copy(bc_hbm.at[wid], bc_v)
    pltpu.sync_copy(bt_hbm.at[wid], bt_v)
    pltpu.sync_copy(co_hbm.at[wid], co_v)
    pltpu.sync_copy(we_hbm.at[wid], we_v)

    def fire(c, b):
        rc_vec = ci_v[c, pl.ds(0, CH)]
        rt_vec = ti_v[c, pl.ds(0, CH)]
        for k in range(CH):
            pltpu.async_copy(ev_hbm.at[pl.ds(rc_vec[k], 1)],
                             ce[b].at[pl.ds(k, 1)], sems[b])
            pltpu.async_copy(eu_hbm.at[pl.ds(rt_vec[k], 1)],
                             te[b].at[pl.ds(k, 1)], sems[b])

        return [
            pltpu.async_copy(vb_hbm.at[bc_v.at[c]], cb[b], sems[b]),
            pltpu.async_copy(ub_hbm.at[bt_v.at[c]], tb[b], sems[b]),
        ]

    def drain(b):
        pltpu.make_async_copy(ev_hbm.at[pl.ds(0, CH)], ce[b], sems[b]).wait()
        pltpu.make_async_copy(eu_hbm.at[pl.ds(0, CH)], te[b], sems[b]).wait()

    lane = lax.iota(jnp.int32, L)
    lane16 = lane * L

    def compute(c, b, lacc):
        @pl.loop(0, CH)
        def _(k):
            p = jnp.zeros((L,), jnp.float32)
            for j in range(D // L):
                s = pl.ds(j * L, L)
                p = p + ce[b][k, s] * te[b][k, s]
            p_v[pl.ds(k * L, L)] = p

        ip = jnp.zeros((L,), jnp.float32)
        for j in range(L):
            ip = ip + plsc.load_gather(p_v, [lane16 + j])

        rc = ci_v[c, pl.ds(0, L)]
        rt = ti_v[c, pl.ds(0, L)]
        cbv = plsc.load_gather(cb[b], [lane, rc & 127])
        tbv = plsc.load_gather(tb[b], [lane, rt & 127])
        cov = co_v[pl.ds(c * CH, L)]
        wev = we_v[pl.ds(c * CH, L)]
        e = ip + cbv + tbv - cov
        return lacc + wev * e * e

    cps = fire(0, 0)
    lacc = jnp.zeros((L,), jnp.float32)
    for c in range(NCHUNK):
        b = c % 2
        if c + 1 < NCHUNK:
            ncps = fire(c + 1, (c + 1) % 2)
        else:
            ncps = None
        for cp in cps:
            cp.wait()
        drain(b)
        lacc = compute(c, b, lacc)
        cps = ncps

    for i in range(128 // L):
        tot_v[pl.ds(i * L, L)] = lacc if i == 0 else jnp.zeros((L,), jnp.float32)
    pltpu.sync_copy(tot_v, red_shared.at[sid])
    plsc.subcore_barrier()

    @pl.when(sid == 0)
    def _():
        pltpu.sync_copy(red_shared, red_v)
        s = jnp.zeros((L,), jnp.float32)
        for i in range(NS):
            s = s + red_v[i, pl.ds(0, L)]
        total = jnp.sum(s)
        for i in range(128 // L):
            tot_v[pl.ds(i * L, L)] = jnp.full((L,), total, dtype=jnp.float32)
        pltpu.sync_copy(tot_v, out_hbm.at[cid])


_cp = pltpu.CompilerParams()
if "needs_layout_passes" in pltpu.CompilerParams.__dataclass_fields__:
    _cp = dataclasses.replace(_cp, needs_layout_passes=False)

_glove_call = functools.partial(
    pl.kernel,
    compiler_params=_cp,
    out_type=jax.ShapeDtypeStruct((NC, 128), jnp.float32),
    mesh=plsc.VectorSubcoreMesh(core_axis_name="c", subcore_axis_name="s"),
    scratch_types=[
        pltpu.VMEM((NCHUNK, CH), jnp.int32),
        pltpu.VMEM((NCHUNK, CH), jnp.int32),
        pltpu.VMEM((NCHUNK, CH), jnp.int32),
        pltpu.VMEM((NCHUNK, CH), jnp.int32),
        pltpu.VMEM((BPW,), jnp.float32),
        pltpu.VMEM((BPW,), jnp.float32),
        pltpu.VMEM((CH, D), jnp.float32),
        pltpu.VMEM((CH, D), jnp.float32),
        pltpu.VMEM((CH, D), jnp.float32),
        pltpu.VMEM((CH, D), jnp.float32),
        pltpu.VMEM((CH, 128), jnp.float32),
        pltpu.VMEM((CH, 128), jnp.float32),
        pltpu.VMEM((CH, 128), jnp.float32),
        pltpu.VMEM((CH, 128), jnp.float32),
        pltpu.VMEM((CH * L,), jnp.float32),
        pltpu.VMEM_SHARED((NS, 128), jnp.float32),
        pltpu.VMEM((NS, 128), jnp.float32),
        pltpu.VMEM((128,), jnp.float32),
        pltpu.SemaphoreType.DMA,
        pltpu.SemaphoreType.DMA,
    ],
)(_glove_body)


def kernel(center_words, target_words, coocs, weighting, emb_v, emb_u,
           v_bias, u_bias):
    ci = center_words.reshape(NW, NCHUNK, CH).astype(jnp.int32)
    ti = target_words.reshape(NW, NCHUNK, CH).astype(jnp.int32)
    bc = ci >> 7
    bt = ti >> 7
    co = coocs.reshape(NW, BPW)
    we = weighting.reshape(NW, BPW)
    vb = jnp.pad(v_bias.reshape(-1), (0, VB_ROWS * 128 - V)).reshape(VB_ROWS, 128)
    ub = jnp.pad(u_bias.reshape(-1), (0, VB_ROWS * 128 - V)).reshape(VB_ROWS, 128)
    out = _glove_call(ci, ti, bc, bt, co, we, emb_v, emb_u, vb, ub)
    return out[0, 0] + out[1, 0]


# speedup vs baseline: 1.3879x; 1.0081x over previous
import dataclasses
import functools

import jax
import jax.numpy as jnp
from jax import lax
from jax.experimental import pallas as pl
from jax.experimental.pallas import tpu as pltpu
from jax.experimental.pallas import tpu_sc as plsc

B = 16384
D = 64
V = 1000000
NC = 2
NS = 16
L = 16
NW = NC * NS
BPW = B // NW
CH = 16
NCHUNK = BPW // CH
VB_ROWS = 7816


def _glove_body(ci_hbm, ti_hbm, bc_hbm, bt_hbm,
                co_hbm, we_hbm, ev_hbm, eu_hbm, vb_hbm,
                ub_hbm, out_hbm,
                ci_v, ti_v, bc_v, bt_v, co_v, we_v,
                ce0, ce1, ce2, ce3, te0, te1, te2, te3,
                cb0, cb1, cb2, cb3, tb0, tb1, tb2, tb3, p_v,
                red_shared, red_v, tot_v, sem0, sem1, sem2, sem3):
    cid = lax.axis_index("c")
    sid = lax.axis_index("s")
    wid = cid * NS + sid

    ce = (ce0, ce1, ce2, ce3)
    te = (te0, te1, te2, te3)
    cb = (cb0, cb1, cb2, cb3)
    tb = (tb0, tb1, tb2, tb3)
    sems = (sem0, sem1, sem2, sem3)

    pltpu.sync_copy(ci_hbm.at[wid], ci_v)
    pltpu.sync_copy(ti_hbm.at[wid], ti_v)
    pltpu.sync_copy(bc_hbm.at[wid], bc_v)
    pltpu.sync_copy(bt_hbm.at[wid], bt_v)
    pltpu.sync_copy(co_hbm.at[wid], co_v)
    pltpu.sync_copy(we_hbm.at[wid], we_v)

    def fire(c, b):
        rc_vec = ci_v[c, pl.ds(0, CH)]
        rt_vec = ti_v[c, pl.ds(0, CH)]
        for k in range(CH):
            pltpu.async_copy(ev_hbm.at[pl.ds(rc_vec[k], 1)],
                             ce[b].at[pl.ds(k, 1)], sems[b])
            pltpu.async_copy(eu_hbm.at[pl.ds(rt_vec[k], 1)],
                             te[b].at[pl.ds(k, 1)], sems[b])

        return [
            pltpu.async_copy(vb_hbm.at[bc_v.at[c]], cb[b], sems[b]),
            pltpu.async_copy(ub_hbm.at[bt_v.at[c]], tb[b], sems[b]),
        ]

    def drain(b):
        pltpu.make_async_copy(ev_hbm.at[pl.ds(0, CH)], ce[b], sems[b]).wait()
        pltpu.make_async_copy(eu_hbm.at[pl.ds(0, CH)], te[b], sems[b]).wait()

    lane = lax.iota(jnp.int32, L)
    lane16 = lane * L

    def compute(c, b, lacc):
        @pl.loop(0, CH)
        def _(k):
            p = jnp.zeros((L,), jnp.float32)
            for j in range(D // L):
                s = pl.ds(j * L, L)
                p = p + ce[b][k, s] * te[b][k, s]
            p_v[pl.ds(k * L, L)] = p

        ip = jnp.zeros((L,), jnp.float32)
        for j in range(L):
            ip = ip + plsc.load_gather(p_v, [lane16 + j])

        rc = ci_v[c, pl.ds(0, L)]
        rt = ti_v[c, pl.ds(0, L)]
        cbv = plsc.load_gather(cb[b], [lane, rc & 127])
        tbv = plsc.load_gather(tb[b], [lane, rt & 127])
        cov = co_v[pl.ds(c * CH, L)]
        wev = we_v[pl.ds(c * CH, L)]
        e = ip + cbv + tbv - cov
        return lacc + wev * e * e

    NBUF = 4
    pend = [fire(c, c % NBUF) for c in range(NBUF - 1)]
    lacc = jnp.zeros((L,), jnp.float32)
    for c in range(NCHUNK):
        b = c % NBUF
        if c + NBUF - 1 < NCHUNK:
            pend.append(fire(c + NBUF - 1, (c + NBUF - 1) % NBUF))
        for cp in pend.pop(0):
            cp.wait()
        drain(b)
        lacc = compute(c, b, lacc)

    for i in range(128 // L):
        tot_v[pl.ds(i * L, L)] = lacc if i == 0 else jnp.zeros((L,), jnp.float32)
    pltpu.sync_copy(tot_v, red_shared.at[sid])
    plsc.subcore_barrier()

    @pl.when(sid == 0)
    def _():
        pltpu.sync_copy(red_shared, red_v)
        s = jnp.zeros((L,), jnp.float32)
        for i in range(NS):
            s = s + red_v[i, pl.ds(0, L)]
        total = jnp.sum(s)
        for i in range(128 // L):
            tot_v[pl.ds(i * L, L)] = jnp.full((L,), total, dtype=jnp.float32)
        pltpu.sync_copy(tot_v, out_hbm.at[cid])


_cp = pltpu.CompilerParams()
if "needs_layout_passes" in pltpu.CompilerParams.__dataclass_fields__:
    _cp = dataclasses.replace(_cp, needs_layout_passes=False)

_glove_call = functools.partial(
    pl.kernel,
    compiler_params=_cp,
    out_type=jax.ShapeDtypeStruct((NC, 128), jnp.float32),
    mesh=plsc.VectorSubcoreMesh(core_axis_name="c", subcore_axis_name="s"),
    scratch_types=[
        pltpu.VMEM((NCHUNK, CH), jnp.int32),
        pltpu.VMEM((NCHUNK, CH), jnp.int32),
        pltpu.VMEM((NCHUNK, CH), jnp.int32),
        pltpu.VMEM((NCHUNK, CH), jnp.int32),
        pltpu.VMEM((BPW,), jnp.float32),
        pltpu.VMEM((BPW,), jnp.float32),
        pltpu.VMEM((CH, D), jnp.float32),
        pltpu.VMEM((CH, D), jnp.float32),
        pltpu.VMEM((CH, D), jnp.float32),
        pltpu.VMEM((CH, D), jnp.float32),
        pltpu.VMEM((CH, D), jnp.float32),
        pltpu.VMEM((CH, D), jnp.float32),
        pltpu.VMEM((CH, D), jnp.float32),
        pltpu.VMEM((CH, D), jnp.float32),
        pltpu.VMEM((CH, 128), jnp.float32),
        pltpu.VMEM((CH, 128), jnp.float32),
        pltpu.VMEM((CH, 128), jnp.float32),
        pltpu.VMEM((CH, 128), jnp.float32),
        pltpu.VMEM((CH, 128), jnp.float32),
        pltpu.VMEM((CH, 128), jnp.float32),
        pltpu.VMEM((CH, 128), jnp.float32),
        pltpu.VMEM((CH, 128), jnp.float32),
        pltpu.VMEM((CH * L,), jnp.float32),
        pltpu.VMEM_SHARED((NS, 128), jnp.float32),
        pltpu.VMEM((NS, 128), jnp.float32),
        pltpu.VMEM((128,), jnp.float32),
        pltpu.SemaphoreType.DMA,
        pltpu.SemaphoreType.DMA,
        pltpu.SemaphoreType.DMA,
        pltpu.SemaphoreType.DMA,
    ],
)(_glove_body)


def kernel(center_words, target_words, coocs, weighting, emb_v, emb_u,
           v_bias, u_bias):
    ci = center_words.reshape(NW, NCHUNK, CH).astype(jnp.int32)
    ti = target_words.reshape(NW, NCHUNK, CH).astype(jnp.int32)
    bc = ci >> 7
    bt = ti >> 7
    co = coocs.reshape(NW, BPW)
    we = weighting.reshape(NW, BPW)
    vb = jnp.pad(v_bias.reshape(-1), (0, VB_ROWS * 128 - V)).reshape(VB_ROWS, 128)
    ub = jnp.pad(u_bias.reshape(-1), (0, VB_ROWS * 128 - V)).reshape(VB_ROWS, 128)
    out = _glove_call(ci, ti, bc, bt, co, we, emb_v, emb_u, vb, ub)
    return out[0, 0] + out[1, 0]
